# Initial kernel scaffold; baseline (speedup 1.0000x reference)
import jax
import jax.numpy as jnp
from jax.experimental import pallas as pl
from jax.experimental.pallas import tpu as pltpu


def _up2_body(x_ref, o_ref):
    x = x_ref[...]
    y = jnp.repeat(x, 2, axis=2)
    o_ref[...] = jnp.repeat(y, 2, axis=1)


def kernel(x):
    n, c, h, w = x.shape
    bc = n * c
    nh, nw = 2 * h, 2 * w

    xf = x.reshape(bc, h, w)
    bt = min(bc, 256)
    out = pl.pallas_call(
        _up2_body,
        out_shape=jax.ShapeDtypeStruct((bc, nh, nw), x.dtype),
        grid=(pl.cdiv(bc, bt),),
        in_specs=[pl.BlockSpec((bt, h, w), lambda i: (i, 0, 0))],
        out_specs=pl.BlockSpec((bt, nh, nw), lambda i: (i, 0, 0)),
        compiler_params=pltpu.CompilerParams(
            dimension_semantics=("parallel",),
            vmem_limit_bytes=64 * 1024 * 1024,
        ),
    )(xf)
    return out.reshape(n, c, nh, nw)


# speedup vs baseline: 1.0251x; 1.0251x over previous
import jax
import jax.numpy as jnp
from jax import lax
from jax.experimental import pallas as pl
from jax.experimental.pallas import tpu as pltpu


def _up2_body(x_ref, o_ref, *, w):
    x = x_ref[...]
    bt, ci, _ = x.shape
    co = 4 * ci
    c = lax.broadcasted_iota(jnp.int32, (bt, co, 128), 1)
    l = lax.broadcasted_iota(jnp.int32, (bt, co, 128), 2)
    xe = jnp.take_along_axis(x, c // 4, axis=1)
    src = 32 * (c % 4) + (l // (4 * w)) * w + (l % (2 * w)) // 2
    o_ref[...] = jnp.take_along_axis(xe, src, axis=2)


def kernel(x):
    n, ch, h, w = x.shape
    bc = n * ch
    hw = h * w
    assert hw % 128 == 0 and 128 % (2 * w) == 0 and 32 % w == 0
    ci = hw // 128

    xf = x.reshape(bc, ci, 128)
    bt = min(bc, 256)
    import functools
    out = pl.pallas_call(
        functools.partial(_up2_body, w=w),
        out_shape=jax.ShapeDtypeStruct((bc, 4 * ci, 128), x.dtype),
        grid=(pl.cdiv(bc, bt),),
        in_specs=[pl.BlockSpec((bt, ci, 128), lambda i: (i, 0, 0))],
        out_specs=pl.BlockSpec((bt, 4 * ci, 128), lambda i: (i, 0, 0)),
        compiler_params=pltpu.CompilerParams(
            dimension_semantics=("parallel",),
            vmem_limit_bytes=48 * 1024 * 1024,
        ),
    )(xf)
    return out.reshape(n, ch, 2 * h, 2 * w)


# speedup vs baseline: 5.1036x; 4.9789x over previous
import jax
import jax.numpy as jnp
from jax import lax
from jax.experimental import pallas as pl
from jax.experimental.pallas import tpu as pltpu


def _up2_nhwc_body(x_ref, o_ref):
    x = x_ref[0]
    h, wc, c = x.shape
    r = lax.broadcasted_iota(jnp.int32, (h, 2 * wc, c), 1)
    xw = jnp.take_along_axis(x, r // 2, axis=1)
    o = jnp.broadcast_to(xw[:, None], (h, 2, 2 * wc, c))
    o_ref[0] = o.reshape(2 * h, 2 * wc, c)


def kernel(x):
    n, c, h, w = x.shape
    assert w % 8 == 0 and c % 128 == 0
    xt = jnp.transpose(x, (0, 2, 3, 1))
    wb = w // 8
    out = pl.pallas_call(
        _up2_nhwc_body,
        out_shape=jax.ShapeDtypeStruct((n, 2 * h, 2 * w, c), x.dtype),
        grid=(n, wb),
        in_specs=[pl.BlockSpec((1, h, 8, c), lambda i, k: (i, 0, k, 0))],
        out_specs=pl.BlockSpec((1, 2 * h, 16, c), lambda i, k: (i, 0, k, 0)),
        compiler_params=pltpu.CompilerParams(
            dimension_semantics=("parallel", "parallel"),
            vmem_limit_bytes=48 * 1024 * 1024,
        ),
    )(xt)
    return jnp.transpose(out, (0, 3, 1, 2))


# speedup vs baseline: 9.5617x; 1.8735x over previous
import jax
import jax.numpy as jnp
from jax import lax
from jax.experimental import pallas as pl
from jax.experimental.pallas import tpu as pltpu


def _up2_nhwc_body(x_ref, o_ref):
    x = x_ref[0]
    h, w, c = x.shape
    r = lax.broadcasted_iota(jnp.int32, (h, 16, c), 1)
    for k in range(w // 8):
        xk = x[:, 8 * k:8 * k + 8, :]
        xw = jnp.take_along_axis(xk, r // 2, axis=1)
        o = jnp.broadcast_to(xw[:, None], (h, 2, 16, c))
        o_ref[0, :, 16 * k:16 * k + 16, :] = o.reshape(2 * h, 16, c)


def kernel(x):
    n, c, h, w = x.shape
    assert w % 8 == 0 and c % 128 == 0
    xt = jnp.transpose(x, (0, 2, 3, 1))
    out = pl.pallas_call(
        _up2_nhwc_body,
        out_shape=jax.ShapeDtypeStruct((n, 2 * h, 2 * w, c), x.dtype),
        grid=(n,),
        in_specs=[pl.BlockSpec((1, h, w, c), lambda i: (i, 0, 0, 0))],
        out_specs=pl.BlockSpec((1, 2 * h, 2 * w, c), lambda i: (i, 0, 0, 0)),
        compiler_params=pltpu.CompilerParams(
            dimension_semantics=("parallel",),
            vmem_limit_bytes=48 * 1024 * 1024,
        ),
    )(xt)
    return jnp.transpose(out, (0, 3, 1, 2))


# speedup vs baseline: 10.1439x; 1.0609x over previous
import jax
import jax.numpy as jnp
from jax import lax
from jax.experimental import pallas as pl
from jax.experimental.pallas import tpu as pltpu


def _up2_nhwc_body(x_ref, o_ref):
    bn, h, w, c = x_ref.shape
    r = lax.broadcasted_iota(jnp.int32, (h, 16, c), 1)
    for b in range(bn):
        x = x_ref[b]
        for k in range(w // 8):
            xk = x[:, 8 * k:8 * k + 8, :]
            xw = jnp.take_along_axis(xk, r // 2, axis=1)
            o = jnp.broadcast_to(xw[:, None], (h, 2, 16, c))
            o_ref[b, :, 16 * k:16 * k + 16, :] = o.reshape(2 * h, 16, c)


def kernel(x):
    n, c, h, w = x.shape
    assert w % 8 == 0 and c % 128 == 0
    xt = jnp.transpose(x, (0, 2, 3, 1))
    bn = 2 if n % 2 == 0 else 1
    out = pl.pallas_call(
        _up2_nhwc_body,
        out_shape=jax.ShapeDtypeStruct((n, 2 * h, 2 * w, c), x.dtype),
        grid=(n // bn,),
        in_specs=[pl.BlockSpec((bn, h, w, c), lambda i: (i, 0, 0, 0))],
        out_specs=pl.BlockSpec((bn, 2 * h, 2 * w, c), lambda i: (i, 0, 0, 0)),
        compiler_params=pltpu.CompilerParams(
            dimension_semantics=("parallel",),
            vmem_limit_bytes=48 * 1024 * 1024,
        ),
    )(xt)
    return jnp.transpose(out, (0, 3, 1, 2))


# speedup vs baseline: 10.4431x; 1.0295x over previous
import jax
import jax.numpy as jnp
from jax import lax
from jax.experimental import pallas as pl
from jax.experimental.pallas import tpu as pltpu


def _up2_nhwc_body(x_ref, o_ref):
    bn, h, w, c = x_ref.shape
    r = lax.broadcasted_iota(jnp.int32, (h, 16, c), 1)
    for b in range(bn):
        x = x_ref[b]
        for k in range(w // 8):
            xk = x[:, 8 * k:8 * k + 8, :]
            xw = jnp.take_along_axis(xk, r // 2, axis=1)
            o = jnp.broadcast_to(xw[:, None], (h, 2, 16, c))
            o_ref[b, :, 16 * k:16 * k + 16, :] = o.reshape(2 * h, 16, c)


def kernel(x):
    n, c, h, w = x.shape
    assert w % 8 == 0 and c % 128 == 0
    xt = jnp.transpose(x, (0, 2, 3, 1))
    bn = 4 if n % 4 == 0 else (2 if n % 2 == 0 else 1)
    out = pl.pallas_call(
        _up2_nhwc_body,
        out_shape=jax.ShapeDtypeStruct((n, 2 * h, 2 * w, c), x.dtype),
        grid=(n // bn,),
        in_specs=[pl.BlockSpec((bn, h, w, c), lambda i: (i, 0, 0, 0))],
        out_specs=pl.BlockSpec((bn, 2 * h, 2 * w, c), lambda i: (i, 0, 0, 0)),
        compiler_params=pltpu.CompilerParams(
            dimension_semantics=("parallel",),
            vmem_limit_bytes=48 * 1024 * 1024,
        ),
    )(xt)
    return jnp.transpose(out, (0, 3, 1, 2))
